# baseline (speedup 1.0000x reference)
import functools

import jax
import jax.numpy as jnp
from jax import lax
from jax.experimental import pallas as pl
from jax.experimental.pallas import tpu as pltpu
from jax.experimental.pallas import tpu_sc as plsc

_VOCAB = 100000
_D = 64
_B = 4096
_L = 50
_N = _B * _L
_CHUNK = 64
_NC = 2
_NS = 16
_NW = _NC * _NS
_PER_W = _N // _NW
_NCHUNK = _PER_W // _CHUNK
_S = 4
_MAIN = (_NCHUNK // _S) * _S
_OUTER = _MAIN // _S


@functools.partial(
    pl.kernel,
    mesh=plsc.VectorSubcoreMesh(core_axis_name="c", subcore_axis_name="s"),
    out_type=jax.ShapeDtypeStruct((_N, 2 * _D), jnp.float32),
    scratch_types=[
        pltpu.VMEM((1, _NCHUNK, _CHUNK), jnp.int32),
        pltpu.VMEM((1, _NCHUNK, _CHUNK), jnp.int32),
        pltpu.VMEM((_S, _CHUNK, 2 * _D), jnp.float32),
        pltpu.VMEM((_S, _CHUNK, 2 * _D), jnp.float32),
        pltpu.SemaphoreType.DMA,
        pltpu.SemaphoreType.DMA,
        pltpu.SemaphoreType.DMA,
        pltpu.SemaphoreType.DMA,
        pltpu.SemaphoreType.DMA,
        pltpu.SemaphoreType.DMA,
        pltpu.SemaphoreType.DMA,
        pltpu.SemaphoreType.DMA,
    ],
)
def _emb_gather(wc_hbm, idx0_hbm, idx1_hbm, out_hbm,
                idx0_v, idx1_v, stag0, stag1,
                gsem0, gsem1, gsem2, gsem3, wsem0, wsem1, wsem2, wsem3):
    wid = lax.axis_index("s") * _NC + lax.axis_index("c")
    base_row = pl.multiple_of(wid * _PER_W, _PER_W)
    gsems = (gsem0, gsem1, gsem2, gsem3)
    wsems = (wsem0, wsem1, wsem2, wsem3)

    pltpu.sync_copy(idx0_hbm.at[pl.ds(wid, 1)], idx0_v)
    pltpu.sync_copy(idx1_hbm.at[pl.ds(wid, 1)], idx1_v)

    def fire_gathers(ch, b):
        pltpu.async_copy(wc_hbm.at[idx0_v.at[0, ch]], stag0.at[b], gsems[b])
        pltpu.async_copy(wc_hbm.at[idx1_v.at[0, ch]], stag1.at[b], gsems[b])

    def wait_gathers(ch, b):
        pltpu.make_async_copy(
            wc_hbm.at[idx0_v.at[0, ch]], stag0.at[b], gsems[b]).wait()
        pltpu.make_async_copy(
            wc_hbm.at[idx1_v.at[0, ch]], stag1.at[b], gsems[b]).wait()

    def merge(b):
        def merge_row(r, c):
            for rr in range(2):
                for k in range(_D // 16):
                    stag0[b, 2 * r + rr, pl.ds(_D + k * 16, 16)] = (
                        stag1[b, 2 * r + rr, pl.ds(_D + k * 16, 16)])
            return c
        lax.fori_loop(0, _CHUNK // 2, merge_row, 0)

    def out_slice(ch):
        row = pl.multiple_of(base_row + ch * _CHUNK, _CHUNK)
        return out_hbm.at[pl.ds(row, _CHUNK)]

    def fire_write(ch, b):
        pltpu.async_copy(stag0.at[b], out_slice(ch), wsems[b])

    def wait_write(ch, b):
        pltpu.make_async_copy(stag0.at[b], out_slice(ch), wsems[b]).wait()

    for b in range(_S):
        fire_gathers(b, b)

    def body(j, carry):
        for b in range(_S):
            ch = j * _S + b
            wait_gathers(ch, b)
            merge(b)
            fire_write(ch, b)
        for b in range(_S):
            ch_next = (j + 1) * _S + b

            @pl.when(ch_next < _NCHUNK)
            def _(b=b, ch_next=ch_next, j=j):
                wait_write(j * _S + b, b)
                fire_gathers(ch_next, b)
        return carry

    lax.fori_loop(0, _OUTER, body, 0)

    for b in range(_NCHUNK - _MAIN):
        ch = _MAIN + b
        wait_gathers(ch, b)
        merge(b)
        fire_write(ch, b)
        wait_write(ch, b)
    for b in range(_NCHUNK - _MAIN, _S):
        wait_write(_MAIN - _S + b, b)


def kernel(X, W0, W1):
    W0p = jnp.pad(W0, ((0, 0), (0, _D)))
    Wc = jax.lax.dynamic_update_slice(W0p, W1, (0, _D))
    idx = jnp.nan_to_num(X).astype(jnp.int32)
    idx0 = idx[:, :, 0].T.reshape(_NW, _NCHUNK, _CHUNK)
    idx1 = idx[:, :, 1].T.reshape(_NW, _NCHUNK, _CHUNK)
    out = _emb_gather(Wc, idx0, idx1)
    return out.reshape(_L, _B, 2 * _D).transpose(1, 0, 2)


# speedup vs baseline: 2.1930x; 2.1930x over previous
import functools

import jax
import jax.numpy as jnp
from jax import lax
from jax.experimental import pallas as pl
from jax.experimental.pallas import tpu as pltpu
from jax.experimental.pallas import tpu_sc as plsc

_VOCAB = 100000
_D = 64
_B = 4096
_L = 50
_N = _B * _L
_CHUNK = 64
_NC = 2
_NS = 16
_NW = _NC * _NS
_PER_W = _N // _NW
_NCHUNK = _PER_W // _CHUNK
_S = 4
_MAIN = (_NCHUNK // _S) * _S
_OUTER = _MAIN // _S


@functools.partial(
    pl.kernel,
    mesh=plsc.VectorSubcoreMesh(core_axis_name="c", subcore_axis_name="s"),
    out_type=jax.ShapeDtypeStruct((_N, 2 * _D), jnp.float32),
    scratch_types=[
        pltpu.VMEM((1, _NCHUNK, _CHUNK), jnp.int32),
        pltpu.VMEM((1, _NCHUNK, _CHUNK), jnp.int32),
        pltpu.VMEM((_S, _CHUNK, 2 * _D), jnp.float32),
        pltpu.VMEM((_S, _CHUNK, 2 * _D), jnp.float32),
        pltpu.SemaphoreType.DMA,
        pltpu.SemaphoreType.DMA,
        pltpu.SemaphoreType.DMA,
        pltpu.SemaphoreType.DMA,
        pltpu.SemaphoreType.DMA,
        pltpu.SemaphoreType.DMA,
        pltpu.SemaphoreType.DMA,
        pltpu.SemaphoreType.DMA,
    ],
)
def _emb_gather(wc_hbm, idx0_hbm, idx1_hbm, out_hbm,
                idx0_v, idx1_v, stag0, stag1,
                gsem0, gsem1, gsem2, gsem3, wsem0, wsem1, wsem2, wsem3):
    wid = lax.axis_index("s") * _NC + lax.axis_index("c")
    base_row = pl.multiple_of(wid * _PER_W, _PER_W)
    gsems = (gsem0, gsem1, gsem2, gsem3)
    wsems = (wsem0, wsem1, wsem2, wsem3)

    pltpu.sync_copy(idx0_hbm.at[pl.ds(wid, 1)], idx0_v)
    pltpu.sync_copy(idx1_hbm.at[pl.ds(wid, 1)], idx1_v)

    def fire_gathers(ch, b):
        pltpu.async_copy(wc_hbm.at[idx0_v.at[0, ch]], stag0.at[b], gsems[b])
        pltpu.async_copy(wc_hbm.at[idx1_v.at[0, ch]], stag1.at[b], gsems[b])

    def wait_gathers(ch, b):
        pltpu.make_async_copy(
            wc_hbm.at[idx0_v.at[0, ch]], stag0.at[b], gsems[b]).wait()
        pltpu.make_async_copy(
            wc_hbm.at[idx1_v.at[0, ch]], stag1.at[b], gsems[b]).wait()

    def merge(b):
        def merge_row(r, c):
            for rr in range(2):
                for k in range(_D // 16):
                    stag0[b, 2 * r + rr, pl.ds(_D + k * 16, 16)] = (
                        stag1[b, 2 * r + rr, pl.ds(_D + k * 16, 16)])
            return c
        lax.fori_loop(0, _CHUNK // 2, merge_row, 0)

    def out_slice(ch):
        row = pl.multiple_of(base_row + ch * _CHUNK, _CHUNK)
        return out_hbm.at[pl.ds(row, _CHUNK)]

    def fire_write(ch, b):
        pltpu.async_copy(stag0.at[b], out_slice(ch), wsems[b])

    def wait_write(ch, b):
        pltpu.make_async_copy(stag0.at[b], out_slice(ch), wsems[b]).wait()

    for b in range(_S):
        fire_gathers(b, b)

    def body(j, carry):
        for b in range(_S):
            ch = j * _S + b
            wait_gathers(ch, b)
            merge(b)
            fire_write(ch, b)
        for b in range(_S):
            ch_next = (j + 1) * _S + b

            @pl.when(ch_next < _NCHUNK)
            def _(b=b, ch_next=ch_next, j=j):
                wait_write(j * _S + b, b)
                fire_gathers(ch_next, b)
        return carry

    lax.fori_loop(0, _OUTER, body, 0)

    for b in range(_NCHUNK - _MAIN):
        ch = _MAIN + b
        wait_gathers(ch, b)
        merge(b)
        fire_write(ch, b)
        wait_write(ch, b)
    for b in range(_NCHUNK - _MAIN, _S):
        wait_write(_MAIN - _S + b, b)


def kernel(X, W0, W1):
    WcT = jax.lax.optimization_barrier(
        jnp.concatenate([W0.T, W1.T], axis=0))
    Wc = WcT.T
    idx = jnp.nan_to_num(X).astype(jnp.int32)
    idx0 = idx[:, :, 0].T.reshape(_NW, _NCHUNK, _CHUNK)
    idx1 = idx[:, :, 1].T.reshape(_NW, _NCHUNK, _CHUNK)
    out = _emb_gather(Wc, idx0, idx1)
    return out.reshape(_L, _B, 2 * _D).transpose(1, 0, 2)


# speedup vs baseline: 2.1960x; 1.0014x over previous
import functools

import jax
import jax.numpy as jnp
from jax import lax
from jax.experimental import pallas as pl
from jax.experimental.pallas import tpu as pltpu
from jax.experimental.pallas import tpu_sc as plsc

_VOCAB = 100000
_D = 64
_B = 4096
_L = 50
_N = _B * _L
_CHUNK = 64
_NC = 2
_NS = 16
_NW = _NC * _NS
_PER_W = _N // _NW
_NCHUNK = _PER_W // _CHUNK
_S = 4
_MAIN = (_NCHUNK // _S) * _S
_OUTER = _MAIN // _S


@functools.partial(
    pl.kernel,
    mesh=plsc.VectorSubcoreMesh(core_axis_name="c", subcore_axis_name="s"),
    out_type=jax.ShapeDtypeStruct((_N, 2 * _D), jnp.float32),
    scratch_types=[
        pltpu.VMEM((1, _NCHUNK, _CHUNK), jnp.int32),
        pltpu.VMEM((1, _NCHUNK, _CHUNK), jnp.int32),
        pltpu.VMEM((_S, _CHUNK, 2 * _D), jnp.float32),
        pltpu.VMEM((_S, _CHUNK, 2 * _D), jnp.float32),
        pltpu.SemaphoreType.DMA,
        pltpu.SemaphoreType.DMA,
        pltpu.SemaphoreType.DMA,
        pltpu.SemaphoreType.DMA,
        pltpu.SemaphoreType.DMA,
        pltpu.SemaphoreType.DMA,
        pltpu.SemaphoreType.DMA,
        pltpu.SemaphoreType.DMA,
    ],
)
def _emb_gather(wc_hbm, idx0_hbm, idx1_hbm, out_hbm,
                idx0_v, idx1_v, stag0, stag1,
                gsem0, gsem1, gsem2, gsem3, wsem0, wsem1, wsem2, wsem3):
    wid = lax.axis_index("s") * _NC + lax.axis_index("c")
    base_row = pl.multiple_of(wid * _PER_W, _PER_W)
    gsems = (gsem0, gsem1, gsem2, gsem3)
    wsems = (wsem0, wsem1, wsem2, wsem3)

    pltpu.sync_copy(idx0_hbm.at[pl.ds(wid, 1)], idx0_v)
    pltpu.sync_copy(idx1_hbm.at[pl.ds(wid, 1)], idx1_v)

    def fire_gathers(ch, b):
        pltpu.async_copy(wc_hbm.at[idx0_v.at[0, ch]], stag0.at[b], gsems[b])
        pltpu.async_copy(wc_hbm.at[idx1_v.at[0, ch]], stag1.at[b], gsems[b])

    def wait_gathers(ch, b):
        pltpu.make_async_copy(
            wc_hbm.at[idx0_v.at[0, ch]], stag0.at[b], gsems[b]).wait()
        pltpu.make_async_copy(
            wc_hbm.at[idx1_v.at[0, ch]], stag1.at[b], gsems[b]).wait()

    def merge(b):
        def merge_row(r, c):
            for rr in range(2):
                for k in range(_D // 16):
                    stag0[b, 2 * r + rr, pl.ds(_D + k * 16, 16)] = (
                        stag1[b, 2 * r + rr, pl.ds(_D + k * 16, 16)])
            return c
        lax.fori_loop(0, _CHUNK // 2, merge_row, 0)

    def out_slice(ch):
        row = pl.multiple_of(base_row + ch * _CHUNK, _CHUNK)
        return out_hbm.at[pl.ds(row, _CHUNK)]

    def fire_write(ch, b):
        pltpu.async_copy(stag0.at[b], out_slice(ch), wsems[b])

    def wait_write(ch, b):
        pltpu.make_async_copy(stag0.at[b], out_slice(ch), wsems[b]).wait()

    for b in range(_S):
        fire_gathers(b, b)

    def body(j, carry):
        for b in range(_S):
            ch = j * _S + b
            wait_gathers(ch, b)
            merge(b)
            fire_write(ch, b)
        for b in range(_S):
            ch_next = (j + 1) * _S + b

            @pl.when(ch_next < _NCHUNK)
            def _(b=b, ch_next=ch_next, j=j):
                wait_write(j * _S + b, b)
                fire_gathers(ch_next, b)
        return carry

    lax.fori_loop(0, _OUTER, body, 0)

    for b in range(_NCHUNK - _MAIN):
        ch = _MAIN + b
        wait_gathers(ch, b)
        merge(b)
        fire_write(ch, b)
        wait_write(ch, b)
    for b in range(_NCHUNK - _MAIN, _S):
        wait_write(_MAIN - _S + b, b)


def kernel(X, W0, W1):
    Wc = jnp.concatenate([W0, W1], axis=1)
    idx = jnp.nan_to_num(X).astype(jnp.int32)
    idx0 = idx[:, :, 0].T.reshape(_NW, _NCHUNK, _CHUNK)
    idx1 = idx[:, :, 1].T.reshape(_NW, _NCHUNK, _CHUNK)
    out = _emb_gather(Wc, idx0, idx1)
    return out.reshape(_L, _B, 2 * _D).transpose(1, 0, 2)
